# Initial kernel scaffold; baseline (speedup 1.0000x reference)
import jax
import jax.numpy as jnp
from jax.experimental import pallas as pl


def kernel(x, w1, b1, w2, b2):
    raise NotImplementedError("write your pallas kernel here")


# speedup vs baseline: 3.1460x; 3.1460x over previous
import jax
import jax.numpy as jnp
from jax.experimental import pallas as pl
from jax.experimental.pallas import tpu as pltpu


def _mlp_kernel(x_ref, w1_ref, b1_ref, w2_ref, b2_ref, o_ref):
    h = jnp.dot(x_ref[...], w1_ref[...], preferred_element_type=jnp.float32)
    h = jnp.maximum(h + b1_ref[...], 0.0)
    out = jnp.dot(h, w2_ref[...], preferred_element_type=jnp.float32)
    o_ref[...] = (out + b2_ref[...]).astype(o_ref.dtype)


def _round_up(n, m):
    return ((n + m - 1) // m) * m


def kernel(x, w1, b1, w2, b2):
    B, A = x.shape
    H = w1.shape[1]
    O = w2.shape[1]

    Ap = max(_round_up(A, 128), 128)
    Hp = max(_round_up(H, 128), 128)
    Op = max(_round_up(O, 128), 128)

    bt = 2048
    Bg = max(_round_up(B, bt), bt)

    xp = x
    if (Bg, Ap) != (B, A):
        xp = jnp.zeros((Bg, Ap), x.dtype).at[:B, :A].set(x)
    w1p = w1
    if (Ap, Hp) != w1.shape:
        w1p = jnp.zeros((Ap, Hp), w1.dtype).at[:A, :H].set(w1)
    w2p = w2
    if (Hp, Op) != w2.shape:
        w2p = jnp.zeros((Hp, Op), w2.dtype).at[:H, :O].set(w2)
    b1p = jnp.zeros((1, Hp), b1.dtype).at[0, :H].set(b1)
    b2p = jnp.zeros((1, Op), b2.dtype).at[0, :O].set(b2)

    flops = 2 * Bg * Ap * Hp + 2 * Bg * Hp * Op
    bytes_accessed = 4 * (Bg * Ap + Ap * Hp + Hp + Hp * Op + Op + Bg * Op)

    outp = pl.pallas_call(
        _mlp_kernel,
        out_shape=jax.ShapeDtypeStruct((Bg, Op), x.dtype),
        grid=(Bg // bt,),
        in_specs=[
            pl.BlockSpec((bt, Ap), lambda i: (i, 0)),
            pl.BlockSpec((Ap, Hp), lambda i: (0, 0)),
            pl.BlockSpec((1, Hp), lambda i: (0, 0)),
            pl.BlockSpec((Hp, Op), lambda i: (0, 0)),
            pl.BlockSpec((1, Op), lambda i: (0, 0)),
        ],
        out_specs=pl.BlockSpec((bt, Op), lambda i: (i, 0)),
        compiler_params=pltpu.CompilerParams(
            dimension_semantics=("parallel",)),
        cost_estimate=pl.CostEstimate(
            flops=flops, transcendentals=0, bytes_accessed=bytes_accessed),
    )(xp, w1p, b1p, w2p, b2p)

    if (Bg, Op) != (B, O):
        outp = outp[:B, :O]
    return outp


# speedup vs baseline: 3.7117x; 1.1798x over previous
import jax
import jax.numpy as jnp
from jax.experimental import pallas as pl
from jax.experimental.pallas import tpu as pltpu


def _mlp_kernel(x_ref, w1_ref, b1_ref, w2_ref, b2_ref, o_ref):
    h = jnp.dot(x_ref[...], w1_ref[...], preferred_element_type=jnp.float32)
    h = jnp.maximum(h + b1_ref[...], 0.0)
    out = jnp.dot(h, w2_ref[...], preferred_element_type=jnp.float32)
    o_ref[...] = (out + b2_ref[...]).astype(o_ref.dtype)


def _round_up(n, m):
    return ((n + m - 1) // m) * m


def kernel(x, w1, b1, w2, b2):
    B, A = x.shape
    H = w1.shape[1]
    O = w2.shape[1]

    Ap = max(_round_up(A, 128), 128)
    Hp = max(_round_up(H, 128), 128)
    Op = max(_round_up(O, 128), 128)

    bt = 4096
    Bg = max(_round_up(B, bt), bt)

    xp = x
    if (Bg, Ap) != (B, A):
        xp = jnp.zeros((Bg, Ap), x.dtype).at[:B, :A].set(x)
    w1p = w1
    if (Ap, Hp) != w1.shape:
        w1p = jnp.zeros((Ap, Hp), w1.dtype).at[:A, :H].set(w1)
    w2p = w2
    if (Hp, Op) != w2.shape:
        w2p = jnp.zeros((Hp, Op), w2.dtype).at[:H, :O].set(w2)
    b1p = jnp.zeros((1, Hp), b1.dtype).at[0, :H].set(b1)
    b2p = jnp.zeros((1, Op), b2.dtype).at[0, :O].set(b2)

    flops = 2 * Bg * Ap * Hp + 2 * Bg * Hp * Op
    bytes_accessed = 4 * (Bg * Ap + Ap * Hp + Hp + Hp * Op + Op + Bg * Op)

    outp = pl.pallas_call(
        _mlp_kernel,
        out_shape=jax.ShapeDtypeStruct((Bg, Op), x.dtype),
        grid=(Bg // bt,),
        in_specs=[
            pl.BlockSpec((bt, Ap), lambda i: (i, 0)),
            pl.BlockSpec((Ap, Hp), lambda i: (0, 0)),
            pl.BlockSpec((1, Hp), lambda i: (0, 0)),
            pl.BlockSpec((Hp, Op), lambda i: (0, 0)),
            pl.BlockSpec((1, Op), lambda i: (0, 0)),
        ],
        out_specs=pl.BlockSpec((bt, Op), lambda i: (i, 0)),
        compiler_params=pltpu.CompilerParams(
            dimension_semantics=("parallel",)),
        cost_estimate=pl.CostEstimate(
            flops=flops, transcendentals=0, bytes_accessed=bytes_accessed),
    )(xp, w1p, b1p, w2p, b2p)

    if (Bg, Op) != (B, O):
        outp = outp[:B, :O]
    return outp


# speedup vs baseline: 3.8138x; 1.0275x over previous
import jax
import jax.numpy as jnp
from jax.experimental import pallas as pl
from jax.experimental.pallas import tpu as pltpu


def _mlp_kernel(x_ref, w1_ref, b1_ref, w2_ref, b2_ref, o_ref):
    h = jnp.dot(x_ref[...], w1_ref[...], preferred_element_type=jnp.float32)
    h = jnp.maximum(h + b1_ref[...], 0.0)
    out = jnp.dot(h, w2_ref[...], preferred_element_type=jnp.float32)
    o_ref[...] = (out + b2_ref[...]).astype(o_ref.dtype)


def _round_up(n, m):
    return ((n + m - 1) // m) * m


def kernel(x, w1, b1, w2, b2):
    B, A = x.shape
    H = w1.shape[1]
    O = w2.shape[1]

    Ap = max(_round_up(A, 128), 128)
    Hp = max(_round_up(H, 128), 128)
    Op = max(_round_up(O, 128), 128)

    bt = 8192
    Bg = max(_round_up(B, bt), bt)

    xp = x
    if (Bg, Ap) != (B, A):
        xp = jnp.zeros((Bg, Ap), x.dtype).at[:B, :A].set(x)
    w1p = w1
    if (Ap, Hp) != w1.shape:
        w1p = jnp.zeros((Ap, Hp), w1.dtype).at[:A, :H].set(w1)
    w2p = w2
    if (Hp, Op) != w2.shape:
        w2p = jnp.zeros((Hp, Op), w2.dtype).at[:H, :O].set(w2)
    b1p = jnp.zeros((1, Hp), b1.dtype).at[0, :H].set(b1)
    b2p = jnp.zeros((1, Op), b2.dtype).at[0, :O].set(b2)

    flops = 2 * Bg * Ap * Hp + 2 * Bg * Hp * Op
    bytes_accessed = 4 * (Bg * Ap + Ap * Hp + Hp + Hp * Op + Op + Bg * Op)

    outp = pl.pallas_call(
        _mlp_kernel,
        out_shape=jax.ShapeDtypeStruct((Bg, Op), x.dtype),
        grid=(Bg // bt,),
        in_specs=[
            pl.BlockSpec((bt, Ap), lambda i: (i, 0)),
            pl.BlockSpec((Ap, Hp), lambda i: (0, 0)),
            pl.BlockSpec((1, Hp), lambda i: (0, 0)),
            pl.BlockSpec((Hp, Op), lambda i: (0, 0)),
            pl.BlockSpec((1, Op), lambda i: (0, 0)),
        ],
        out_specs=pl.BlockSpec((bt, Op), lambda i: (i, 0)),
        compiler_params=pltpu.CompilerParams(
            dimension_semantics=("parallel",)),
        cost_estimate=pl.CostEstimate(
            flops=flops, transcendentals=0, bytes_accessed=bytes_accessed),
    )(xp, w1p, b1p, w2p, b2p)

    if (Bg, Op) != (B, O):
        outp = outp[:B, :O]
    return outp
